# baseline (speedup 1.0000x reference)
import jax
import jax.numpy as jnp
from jax.experimental import pallas as pl

IN_FEATURES = 2048
OUT_FEATURES = 2048
NUM_EXPERTS = 64
TOP_K = 2
RANK = 16
SCALING = 32 / 16

TOKEN_TILE = 512
SUB_TILES = 2


def _moe_lora_kernel(x_ref, wr_ref, a_ref, b_ref, o_ref):
    sub = TOKEN_TILE // SUB_TILES
    for s_i in range(SUB_TILES):
        sl = slice(s_i * sub, (s_i + 1) * sub)
        x = x_ref[sl, :]
        scores = jnp.dot(x, wr_ref[...],
                         preferred_element_type=jnp.float32)

        e_iota = jax.lax.broadcasted_iota(jnp.int32, scores.shape, 1)
        big = jnp.int32(NUM_EXPERTS)
        neg = jnp.float32(-3.0e38)

        s1 = jnp.max(scores, axis=-1, keepdims=True)
        i1 = jnp.min(jnp.where(scores == s1, e_iota, big), axis=-1,
                     keepdims=True)
        m1 = e_iota == i1
        scores2 = jnp.where(m1, neg, scores)
        s2 = jnp.max(scores2, axis=-1, keepdims=True)
        i2 = jnp.min(jnp.where(scores2 == s2, e_iota, big), axis=-1,
                     keepdims=True)

        z = jnp.exp(s2 - s1)
        inv = jnp.float32(SCALING) / (1.0 + z)
        g1 = inv
        g2 = z * inv

        h = jax.lax.dot_general(x, a_ref[...], (((1,), (1,)), ((), ())),
                                preferred_element_type=jnp.float32)

        col_e = jax.lax.shift_right_logical(
            jax.lax.broadcasted_iota(jnp.int32, h.shape, 1), 4)
        grep = jnp.where(col_e == i1, g1,
                         jnp.where(col_e == i2, g2, 0.0))
        hp = h * grep

        o_ref[sl, :] = jnp.dot(hp, b_ref[...],
                               preferred_element_type=jnp.float32)


def kernel(x, A, B, Wr):
    shape_prefix = x.shape[:-1]
    d = IN_FEATURES
    n = 1
    for s in shape_prefix:
        n *= s
    x_flat = x.reshape(n, d)

    er = NUM_EXPERTS * RANK
    a2 = A.reshape(er, d)
    bp = B.transpose(0, 2, 1).reshape(er, OUT_FEATURES)

    t = TOKEN_TILE
    grid = (n // t,)
    out = pl.pallas_call(
        _moe_lora_kernel,
        grid=grid,
        in_specs=[
            pl.BlockSpec((t, d), lambda i: (i, 0)),
            pl.BlockSpec((d, NUM_EXPERTS), lambda i: (0, 0)),
            pl.BlockSpec((er, d), lambda i: (0, 0)),
            pl.BlockSpec((er, OUT_FEATURES), lambda i: (0, 0)),
        ],
        out_specs=pl.BlockSpec((t, OUT_FEATURES), lambda i: (i, 0)),
        out_shape=jax.ShapeDtypeStruct((n, OUT_FEATURES), jnp.float32),
    )(x_flat, Wr, a2, bp)
    return out.reshape(*shape_prefix, OUT_FEATURES)


# speedup vs baseline: 1.0579x; 1.0579x over previous
import jax
import jax.numpy as jnp
from jax.experimental import pallas as pl

IN_FEATURES = 2048
OUT_FEATURES = 2048
NUM_EXPERTS = 64
TOP_K = 2
RANK = 16
SCALING = 32 / 16

TOKEN_TILE = 1024
SUB_TILES = 4


def _moe_lora_kernel(x_ref, wr_ref, a_ref, b_ref, o_ref):
    sub = TOKEN_TILE // SUB_TILES
    greps = []
    hs = []
    for s_i in range(SUB_TILES):
        sl = slice(s_i * sub, (s_i + 1) * sub)
        x = x_ref[sl, :]
        scores = jnp.dot(x, wr_ref[...],
                         preferred_element_type=jnp.float32)

        e_iota = jax.lax.broadcasted_iota(jnp.int32, scores.shape, 1)
        big = jnp.int32(NUM_EXPERTS)
        neg = jnp.float32(-3.0e38)

        s1 = jnp.max(scores, axis=-1, keepdims=True)
        i1 = jnp.min(jnp.where(scores == s1, e_iota, big), axis=-1,
                     keepdims=True)
        m1 = e_iota == i1
        scores2 = jnp.where(m1, neg, scores)
        s2 = jnp.max(scores2, axis=-1, keepdims=True)
        i2 = jnp.min(jnp.where(scores2 == s2, e_iota, big), axis=-1,
                     keepdims=True)

        z = jnp.exp(s2 - s1)
        inv = jnp.float32(SCALING) / (1.0 + z)
        g1 = inv
        g2 = z * inv

        h = jax.lax.dot_general(x, a_ref[...], (((1,), (1,)), ((), ())),
                                preferred_element_type=jnp.float32)
        col_e = jax.lax.shift_right_logical(
            jax.lax.broadcasted_iota(jnp.int32, h.shape, 1), 4)
        grep = jnp.where(col_e == i1, g1,
                         jnp.where(col_e == i2, g2, 0.0))
        hs.append(h)
        greps.append(grep)
    for s_i in range(SUB_TILES):
        sl = slice(s_i * sub, (s_i + 1) * sub)
        hp = hs[s_i] * greps[s_i]
        o_ref[sl, :] = jnp.dot(hp, b_ref[...],
                               preferred_element_type=jnp.float32)


def kernel(x, A, B, Wr):
    shape_prefix = x.shape[:-1]
    d = IN_FEATURES
    n = 1
    for s in shape_prefix:
        n *= s
    x_flat = x.reshape(n, d)

    er = NUM_EXPERTS * RANK
    a2 = A.reshape(er, d)
    bp = B.transpose(0, 2, 1).reshape(er, OUT_FEATURES)

    t = TOKEN_TILE
    grid = (n // t,)
    out = pl.pallas_call(
        _moe_lora_kernel,
        grid=grid,
        in_specs=[
            pl.BlockSpec((t, d), lambda i: (i, 0)),
            pl.BlockSpec((d, NUM_EXPERTS), lambda i: (0, 0)),
            pl.BlockSpec((er, d), lambda i: (0, 0)),
            pl.BlockSpec((er, OUT_FEATURES), lambda i: (0, 0)),
        ],
        out_specs=pl.BlockSpec((t, OUT_FEATURES), lambda i: (i, 0)),
        out_shape=jax.ShapeDtypeStruct((n, OUT_FEATURES), jnp.float32),
    )(x_flat, Wr, a2, bp)
    return out.reshape(*shape_prefix, OUT_FEATURES)
